# baseline (speedup 1.0000x reference)
import functools

import jax
import jax.numpy as jnp
from jax import lax
from jax.experimental import pallas as pl
from jax.experimental.pallas import tpu as pltpu
from jax.experimental.pallas import tpu_sc as plsc

_B = 16384
_L = 20
_EMBED = 64
_V = 1000000
_NW = 32
_BC = 256
_NBC = _B // (_NW * _BC)
_VW = 400
_NWIN = _V // _VW

_mesh = plsc.VectorSubcoreMesh(
    core_axis_name="c", subcore_axis_name="s", num_cores=2, num_subcores=16
)

_params = pltpu.CompilerParams(use_tc_tiling_on_sc=False, needs_layout_passes=False)


@functools.partial(
    pl.kernel,
    out_type=jax.ShapeDtypeStruct((_V, _EMBED), jnp.float32),
    mesh=_mesh,
    scratch_types=[
        pltpu.VMEM((_EMBED, _VW), jnp.float32),
        pltpu.VMEM((_EMBED, _VW), jnp.float32),
        pltpu.VMEM((_VW, _EMBED), jnp.float32),
        pltpu.VMEM((_VW, _EMBED), jnp.float32),
        pltpu.SemaphoreType.DMA,
        pltpu.SemaphoreType.DMA,
        pltpu.SemaphoreType.DMA,
        pltpu.SemaphoreType.DMA,
    ],
    compiler_params=_params,
)
def _transpose_table(wet_hbm, wt_hbm, a0, a1, t0, t1, g0, g1, w0, w1):
    wid = lax.axis_index("s") * 2 + lax.axis_index("c")
    av = [a0, a1]
    tv = [t0, t1]
    gs = [g0, g1]
    ws = [w0, w1]
    lane = lax.broadcasted_iota(jnp.int32, (16,), 0)
    rowsel = [(lane + k) & 15 for k in range(16)]

    pltpu.async_copy(wet_hbm.at[:, pl.ds(wid * _VW, _VW)], av[0], gs[0])

    def win_body(t, carry):
        for s in range(2):
            tt = 2 * t + s
            j = wid + _NW * tt

            @pl.when(j < _NWIN)
            def _():
                pltpu.make_async_copy(
                    wet_hbm.at[:, pl.ds(0, _VW)], av[s], gs[s]
                ).wait()
                jn = j + _NW

                @pl.when(jn < _NWIN)
                def _():
                    pltpu.async_copy(
                        wet_hbm.at[:, pl.ds(jn * _VW, _VW)], av[1 - s], gs[1 - s]
                    )

                @pl.when(tt >= 2)
                def _():
                    pltpu.make_async_copy(
                        tv[s], wt_hbm.at[pl.ds(0, _VW)], ws[s]
                    ).wait()

                def vg_body(vg, c2):
                    colv = lane + vg * 16
                    for er in range(_EMBED // 16):
                        er0 = er * 16
                        for k in range(16):
                            rsel = rowsel[k] + er0
                            vals = plsc.load_gather(av[s], [rsel, colv])
                            plsc.store_scatter(tv[s], [colv, rsel], vals)
                    return c2

                lax.fori_loop(0, _VW // 16, vg_body, 0)
                pltpu.async_copy(tv[s], wt_hbm.at[pl.ds(j * _VW, _VW)], ws[s])
        return carry

    lax.fori_loop(0, (_NWIN // _NW + 2) // 2, win_body, 0)
    for s in range(2):
        pltpu.make_async_copy(tv[s], wt_hbm.at[pl.ds(0, _VW)], ws[s]).wait()


@functools.partial(
    pl.kernel,
    out_type=jax.ShapeDtypeStruct((_L, _EMBED, _B), jnp.float32),
    mesh=_mesh,
    scratch_types=[
        pltpu.VMEM((_L, _BC), jnp.int32),
        pltpu.VMEM((_BC, _EMBED), jnp.float32),
        pltpu.VMEM((_BC, _EMBED), jnp.float32),
        pltpu.VMEM((_BC, _EMBED), jnp.float32),
        pltpu.VMEM((_BC, _EMBED), jnp.float32),
        pltpu.VMEM((_EMBED, _BC), jnp.float32),
        pltpu.VMEM((_EMBED, _BC), jnp.float32),
        pltpu.VMEM((32, _EMBED), jnp.float32),
        pltpu.SemaphoreType.DMA,
        pltpu.SemaphoreType.DMA,
        pltpu.SemaphoreType.DMA,
        pltpu.SemaphoreType.DMA,
        pltpu.SemaphoreType.DMA,
        pltpu.SemaphoreType.DMA,
    ],
    compiler_params=_params,
)
def _emb_lookup(
    xt_hbm, wt_hbm, pemb_hbm, out_hbm,
    idx_v, r0, r1, r2, r3, ob0, ob1, pos_v, g0, g1, g2, g3, w0, w1,
):
    wid = lax.axis_index("s") * 2 + lax.axis_index("c")
    rows = [r0, r1, r2, r3]
    outb = [ob0, ob1]
    gsem = [g0, g1, g2, g3]
    wsem = [w0, w1]
    lane = lax.broadcasted_iota(jnp.int32, (16,), 0)
    rowsel = [(lane + k) & 15 for k in range(16)]
    pltpu.sync_copy(pemb_hbm, pos_v)

    for bc in range(_NBC):
        b0 = wid * _BC + bc * (_NW * _BC)
        pltpu.sync_copy(xt_hbm.at[:, pl.ds(b0, _BC)], idx_v)
        for s in range(3):
            pltpu.async_copy(wt_hbm.at[idx_v.at[s]], rows[s], gsem[s])

        def quad_body(p, carry):
            for q in range(4):
                l = 4 * p + q
                pltpu.make_async_copy(
                    wt_hbm.at[pl.ds(0, _BC)], rows[q], gsem[q]
                ).wait()

                @pl.when(l + 3 < _L)
                def _():
                    pltpu.async_copy(
                        wt_hbm.at[idx_v.at[l + 3]],
                        rows[(q + 3) % 4],
                        gsem[(q + 3) % 4],
                    )

                @pl.when(l >= 2)
                def _():
                    pltpu.make_async_copy(
                        outb[q % 2], out_hbm.at[0, :, pl.ds(b0, _BC)], wsem[q % 2]
                    ).wait()

                splat_l = jnp.broadcast_to(l, (16,))

                def eg_body(eg, c3):
                    col_e = lane + eg * 16
                    pvec = plsc.load_gather(pos_v, [splat_l, col_e])

                    def bg_body(bg, c4):
                        br0 = bg * 16
                        for k in range(16):
                            rsel = rowsel[k] + br0
                            vals = plsc.load_gather(rows[q], [rsel, col_e])
                            plsc.store_scatter(
                                outb[q % 2], [col_e, rsel], vals + pvec
                            )
                        return c4

                    lax.fori_loop(0, _BC // 16, bg_body, 0)
                    return c3

                lax.fori_loop(0, _EMBED // 16, eg_body, 0)
                pltpu.async_copy(
                    outb[q % 2], out_hbm.at[l, :, pl.ds(b0, _BC)], wsem[q % 2]
                )
            return carry

        lax.fori_loop(0, _L // 4, quad_body, 0)
        for s in range(2):
            pltpu.make_async_copy(
                outb[s], out_hbm.at[0, :, pl.ds(b0, _BC)], wsem[s]
            ).wait()


def kernel(x, word_emb, pos_emb):
    wt = _transpose_table(word_emb.T)
    out_t = _emb_lookup(x.T, wt, pos_emb)
    return jnp.transpose(out_t, (2, 0, 1))


# speedup vs baseline: 6.5221x; 6.5221x over previous
import functools

import jax
import jax.numpy as jnp
from jax import lax
from jax.experimental import pallas as pl
from jax.experimental.pallas import tpu as pltpu
from jax.experimental.pallas import tpu_sc as plsc

_B = 16384
_L = 20
_EMBED = 64
_V = 1000000
_NW = 32
_BC = 256
_NBC = _B // (_NW * _BC)
_VW = 400
_NWIN = _V // _VW

_mesh = plsc.VectorSubcoreMesh(
    core_axis_name="c", subcore_axis_name="s", num_cores=2, num_subcores=16
)

_params = pltpu.CompilerParams(use_tc_tiling_on_sc=False, needs_layout_passes=False)


@functools.partial(
    pl.kernel,
    out_type=jax.ShapeDtypeStruct((_V, _EMBED), jnp.float32),
    mesh=_mesh,
    scratch_types=[
        pltpu.VMEM((_EMBED, _VW), jnp.float32),
        pltpu.VMEM((_EMBED, _VW), jnp.float32),
        pltpu.VMEM((_VW, _EMBED), jnp.float32),
        pltpu.VMEM((_VW, _EMBED), jnp.float32),
        pltpu.SemaphoreType.DMA,
        pltpu.SemaphoreType.DMA,
        pltpu.SemaphoreType.DMA,
        pltpu.SemaphoreType.DMA,
    ],
    compiler_params=_params,
)
def _transpose_table(wet_hbm, wt_hbm, a0, a1, t0, t1, g0, g1, w0, w1):
    wid = lax.axis_index("s") * 2 + lax.axis_index("c")
    av = [a0, a1]
    tv = [t0, t1]
    gs = [g0, g1]
    ws = [w0, w1]
    lane = lax.broadcasted_iota(jnp.int32, (16,), 0)
    rowsel = [(lane + k) & 15 for k in range(16)]

    pltpu.async_copy(wet_hbm.at[:, pl.ds(wid * _VW, _VW)], av[0], gs[0])

    def win_body(t, carry):
        for s in range(2):
            tt = 2 * t + s
            j = wid + _NW * tt

            @pl.when(j < _NWIN)
            def _():
                pltpu.make_async_copy(
                    wet_hbm.at[:, pl.ds(0, _VW)], av[s], gs[s]
                ).wait()
                jn = j + _NW

                @pl.when(jn < _NWIN)
                def _():
                    pltpu.async_copy(
                        wet_hbm.at[:, pl.ds(jn * _VW, _VW)], av[1 - s], gs[1 - s]
                    )

                @pl.when(tt >= 2)
                def _():
                    pltpu.make_async_copy(
                        tv[s], wt_hbm.at[pl.ds(0, _VW)], ws[s]
                    ).wait()

                def vg_body(vg, c2):
                    colv = lane + vg * 16
                    for er in range(_EMBED // 16):
                        er0 = er * 16
                        for k in range(16):
                            rsel = rowsel[k] + er0
                            vals = plsc.load_gather(av[s], [rsel, colv])
                            plsc.store_scatter(tv[s], [colv, rsel], vals)
                    return c2

                lax.fori_loop(0, _VW // 16, vg_body, 0)
                pltpu.async_copy(tv[s], wt_hbm.at[pl.ds(j * _VW, _VW)], ws[s])
        return carry

    lax.fori_loop(0, (_NWIN // _NW + 2) // 2, win_body, 0)
    for s in range(2):
        pltpu.make_async_copy(tv[s], wt_hbm.at[pl.ds(0, _VW)], ws[s]).wait()


@functools.partial(
    pl.kernel,
    out_type=jax.ShapeDtypeStruct((_L, _EMBED, _B), jnp.float32),
    mesh=_mesh,
    scratch_types=[
        pltpu.VMEM((_L, _BC), jnp.int32),
        pltpu.VMEM((_BC, _EMBED), jnp.float32),
        pltpu.VMEM((_BC, _EMBED), jnp.float32),
        pltpu.VMEM((_BC, _EMBED), jnp.float32),
        pltpu.VMEM((_BC, _EMBED), jnp.float32),
        pltpu.VMEM((_EMBED, _BC), jnp.float32),
        pltpu.VMEM((_EMBED, _BC), jnp.float32),
        pltpu.VMEM((32, _EMBED), jnp.float32),
        pltpu.SemaphoreType.DMA,
        pltpu.SemaphoreType.DMA,
        pltpu.SemaphoreType.DMA,
        pltpu.SemaphoreType.DMA,
        pltpu.SemaphoreType.DMA,
        pltpu.SemaphoreType.DMA,
    ],
    compiler_params=_params,
)
def _emb_lookup(
    xt_hbm, wt_hbm, pemb_hbm, out_hbm,
    idx_v, r0, r1, r2, r3, ob0, ob1, pos_v, g0, g1, g2, g3, w0, w1,
):
    wid = lax.axis_index("s") * 2 + lax.axis_index("c")
    rows = [r0, r1, r2, r3]
    outb = [ob0, ob1]
    gsem = [g0, g1, g2, g3]
    wsem = [w0, w1]
    lane = lax.broadcasted_iota(jnp.int32, (16,), 0)
    rowsel = [(lane + k) & 15 for k in range(16)]
    pltpu.sync_copy(pemb_hbm, pos_v)

    for bc in range(_NBC):
        b0 = wid * _BC + bc * (_NW * _BC)
        pltpu.sync_copy(xt_hbm.at[:, pl.ds(b0, _BC)], idx_v)
        for s in range(3):
            pltpu.async_copy(wt_hbm.at[idx_v.at[s]], rows[s], gsem[s])

        def quad_body(p, carry):
            for q in range(4):
                l = 4 * p + q
                pltpu.make_async_copy(
                    wt_hbm.at[pl.ds(0, _BC)], rows[q], gsem[q]
                ).wait()

                @pl.when(l + 3 < _L)
                def _():
                    pltpu.async_copy(
                        wt_hbm.at[idx_v.at[l + 3]],
                        rows[(q + 3) % 4],
                        gsem[(q + 3) % 4],
                    )

                @pl.when(l >= 2)
                def _():
                    pltpu.make_async_copy(
                        outb[q % 2], out_hbm.at[0, :, pl.ds(b0, _BC)], wsem[q % 2]
                    ).wait()

                splat_l = jnp.broadcast_to(l, (16,))

                def eg_body(eg, c3):
                    col_e = lane + eg * 16
                    pvec = plsc.load_gather(pos_v, [splat_l, col_e])

                    def bg_body(bg, c4):
                        br0 = bg * 16
                        for k in range(16):
                            rsel = rowsel[k] + br0
                            vals = plsc.load_gather(rows[q], [rsel, col_e])
                            plsc.store_scatter(
                                outb[q % 2], [col_e, rsel], vals + pvec
                            )
                        return c4

                    lax.fori_loop(0, _BC // 16, bg_body, 0)
                    return c3

                lax.fori_loop(0, _EMBED // 16, eg_body, 0)
                pltpu.async_copy(
                    outb[q % 2], out_hbm.at[l, :, pl.ds(b0, _BC)], wsem[q % 2]
                )
            return carry

        lax.fori_loop(0, _L // 4, quad_body, 0)
        for s in range(2):
            pltpu.make_async_copy(
                outb[s], out_hbm.at[0, :, pl.ds(b0, _BC)], wsem[s]
            ).wait()


def kernel(x, word_emb, pos_emb):
    out_t = _emb_lookup(x.T, word_emb, pos_emb)
    return jnp.transpose(out_t, (2, 0, 1))
